# baseline (speedup 1.0000x reference)
import jax
import jax.numpy as jnp
from jax.experimental import pallas as pl
from jax.experimental.pallas import tpu as pltpu

B = 4
SEQ = 1024
H = 8
D = 64
T = B * SEQ
BLOCK_SIZE = 32
NB = SEQ // BLOCK_SIZE
S = 8
TQ = 256
NT = SEQ // TQ
BPT = TQ // BLOCK_SIZE


def _silu(x):
    return x * jax.nn.sigmoid(x)


def _one_head(qm, km, vm, wc, ws):
    scale = D ** (-0.5)

    g_cmp = jax.nn.sigmoid(jnp.sum(qm * wc, axis=1, keepdims=True))
    g_slc = jax.nn.sigmoid(jnp.sum(qm * ws, axis=1, keepdims=True))

    kc = jnp.mean(km.reshape(NB, BLOCK_SIZE, D), axis=1)
    vc = jnp.mean(vm.reshape(NB, BLOCK_SIZE, D), axis=1)

    scT = jax.lax.dot_general(kc, qm, (((1,), (1,)), ((), ())),
                              preferred_element_type=jnp.float32) * scale
    jrow = jax.lax.broadcasted_iota(jnp.int32, (NB, SEQ), 0)
    qblkT = jax.lax.broadcasted_iota(jnp.int32, (NB, SEQ), 1) // BLOCK_SIZE
    causalT = qblkT >= jrow
    smT = jnp.where(causalT, scT, -1e9)

    cur = smT
    for _ in range(S - 1):
        mx = jnp.max(cur, axis=0, keepdims=True)
        cur = jnp.where(cur == mx, -jnp.inf, cur)
    thr = jnp.max(cur, axis=0, keepdims=True)
    mfT = ((smT >= thr) & causalT).astype(jnp.float32)

    p_cmpT = jnp.where(causalT, _silu(scT), 0.0)
    o_cmp = jax.lax.dot_general(p_cmpT, vc, (((0,), (0,)), ((), ())),
                                preferred_element_type=jnp.float32) * g_cmp

    e_row = jax.lax.broadcasted_iota(jnp.int32, (NB, SEQ), 0)
    e_col = jax.lax.broadcasted_iota(jnp.int32, (NB, SEQ), 1) // BLOCK_SIZE
    EF = (e_row == e_col).astype(jnp.float32)
    qpos = jax.lax.broadcasted_iota(jnp.int32, (TQ, TQ), 0)
    kpos = jax.lax.broadcasted_iota(jnp.int32, (TQ, TQ), 1)
    causal_tile = (qpos >= kpos).astype(jnp.float32)

    kb16 = km.astype(jnp.bfloat16)
    vb16 = vm.astype(jnp.bfloat16)
    out_tiles = []
    for ti in range(NT):
        W = (ti + 1) * TQ
        qsl = slice(ti * TQ, (ti + 1) * TQ)
        qt = qm[qsl].astype(jnp.bfloat16)
        s = jax.lax.dot_general(qt, kb16[:W], (((1,), (1,)), ((), ())),
                                preferred_element_type=jnp.float32)
        s = s * scale
        if ti == 0:
            p = _silu(s)
        else:
            mexp = jax.lax.dot_general(
                mfT[:(ti + 1) * BPT, qsl], EF[:(ti + 1) * BPT, :W],
                (((0,), (0,)), ((), ())),
                preferred_element_type=jnp.float32)
            p = _silu(s) * mexp
        pd = p[:, ti * TQ:W] * causal_tile
        p = pd if ti == 0 else jnp.concatenate([p[:, :ti * TQ], pd], axis=1)
        acc = jnp.dot(p.astype(jnp.bfloat16), vb16[:W],
                      preferred_element_type=jnp.float32)
        out_tiles.append(o_cmp[qsl] + acc * g_slc[qsl])
    return jnp.concatenate(out_tiles, axis=0)


def _body(q_ref, k_ref, v_ref, wc_ref, ws_ref, o_ref):
    qf = q_ref[0].reshape(SEQ, H * D)
    kf = k_ref[0].reshape(SEQ, H * D)
    vf = v_ref[0].reshape(SEQ, H * D)
    for h in range(H):
        cs = slice(h * D, (h + 1) * D)
        o_ref[0, :, h, :] = _one_head(
            qf[:, cs], kf[:, cs], vf[:, cs],
            wc_ref[0][None, :], ws_ref[0][None, :])


def kernel(q, k, v, u, x_offsets, Wg_cmp, Wg_slc, Wg_swa):
    qh = q.reshape(B, SEQ, H, D)
    kh = k.reshape(B, SEQ, H, D)
    vh = v.reshape(B, SEQ, H, D)

    out4 = pl.pallas_call(
        _body,
        grid=(B,),
        in_specs=[
            pl.BlockSpec((1, SEQ, H, D), lambda b: (b, 0, 0, 0)),
            pl.BlockSpec((1, SEQ, H, D), lambda b: (b, 0, 0, 0)),
            pl.BlockSpec((1, SEQ, H, D), lambda b: (b, 0, 0, 0)),
            pl.BlockSpec((1, D), lambda b: (0, 0)),
            pl.BlockSpec((1, D), lambda b: (0, 0)),
        ],
        out_specs=pl.BlockSpec((1, SEQ, H, D), lambda b: (b, 0, 0, 0)),
        out_shape=jax.ShapeDtypeStruct((B, SEQ, H, D), jnp.float32),
        compiler_params=pltpu.CompilerParams(
            dimension_semantics=("parallel",)),
    )(qh, kh, vh, Wg_cmp.reshape(1, D), Wg_slc.reshape(1, D))

    return out4.reshape(T, H, D)


# speedup vs baseline: 1.0130x; 1.0130x over previous
import jax
import jax.numpy as jnp
from jax.experimental import pallas as pl
from jax.experimental.pallas import tpu as pltpu

B = 4
SEQ = 1024
H = 8
D = 64
T = B * SEQ
BLOCK_SIZE = 32
NB = SEQ // BLOCK_SIZE
S = 8
TQ = 256
NT = SEQ // TQ
BPT = TQ // BLOCK_SIZE


def _silu(x):
    return x * jax.nn.sigmoid(x)


def _one_head(qm, km, vm, wc, ws):
    scale = D ** (-0.5)

    g_cmp = jax.nn.sigmoid(jnp.sum(qm * wc, axis=1, keepdims=True))
    g_slc = jax.nn.sigmoid(jnp.sum(qm * ws, axis=1, keepdims=True))

    kc = jnp.mean(km.reshape(NB, BLOCK_SIZE, D), axis=1)
    vc = jnp.mean(vm.reshape(NB, BLOCK_SIZE, D), axis=1)

    scT = jax.lax.dot_general(kc, qm, (((1,), (1,)), ((), ())),
                              preferred_element_type=jnp.float32) * scale
    jrow = jax.lax.broadcasted_iota(jnp.int32, (NB, SEQ), 0)
    qblkT = jax.lax.broadcasted_iota(jnp.int32, (NB, SEQ), 1) // BLOCK_SIZE
    causalT = qblkT >= jrow
    smT = jnp.where(causalT, scT, -1e9)

    cur = smT
    for _ in range(S - 1):
        mx = jnp.max(cur, axis=0, keepdims=True)
        cur = jnp.where(cur == mx, -jnp.inf, cur)
    thr = jnp.max(cur, axis=0, keepdims=True)
    mfT = ((smT >= thr) & causalT).astype(jnp.float32)

    p_cmpT = jnp.where(causalT, _silu(scT), 0.0)
    o_cmp = jax.lax.dot_general(p_cmpT, vc, (((0,), (0,)), ((), ())),
                                preferred_element_type=jnp.float32) * g_cmp

    e_row = jax.lax.broadcasted_iota(jnp.int32, (NB, SEQ), 0)
    e_col = jax.lax.broadcasted_iota(jnp.int32, (NB, SEQ), 1) // BLOCK_SIZE
    EF = (e_row == e_col).astype(jnp.float32)
    qpos = jax.lax.broadcasted_iota(jnp.int32, (TQ, TQ), 0)
    kpos = jax.lax.broadcasted_iota(jnp.int32, (TQ, TQ), 1)
    causal_tile = (qpos >= kpos).astype(jnp.float32)

    kb16 = km.astype(jnp.bfloat16)
    vb16 = vm.astype(jnp.bfloat16)
    out_tiles = []
    for ti in range(NT):
        W = (ti + 1) * TQ
        qsl = slice(ti * TQ, (ti + 1) * TQ)
        qt = qm[qsl].astype(jnp.bfloat16)
        s = jax.lax.dot_general(qt, kb16[:W], (((1,), (1,)), ((), ())),
                                preferred_element_type=jnp.float32)
        s = s * scale
        mexp = jax.lax.dot_general(
            mfT[:(ti + 1) * BPT, qsl], EF[:(ti + 1) * BPT, :W],
            (((0,), (0,)), ((), ())),
            preferred_element_type=jnp.float32)
        p = _silu(s) * mexp
        pd = p[:, ti * TQ:W] * causal_tile
        p = pd if ti == 0 else jnp.concatenate([p[:, :ti * TQ], pd], axis=1)
        acc = jnp.dot(p.astype(jnp.bfloat16), vb16[:W],
                      preferred_element_type=jnp.float32)
        out_tiles.append(o_cmp[qsl] + acc * g_slc[qsl])
    return jnp.concatenate(out_tiles, axis=0)


def _body(q_ref, k_ref, v_ref, wc_ref, ws_ref, o_ref):
    qf = q_ref[0].reshape(SEQ, H * D)
    kf = k_ref[0].reshape(SEQ, H * D)
    vf = v_ref[0].reshape(SEQ, H * D)
    for h in range(H):
        cs = slice(h * D, (h + 1) * D)
        o_ref[0, :, h, :] = _one_head(
            qf[:, cs], kf[:, cs], vf[:, cs],
            wc_ref[0][None, :], ws_ref[0][None, :])


def kernel(q, k, v, u, x_offsets, Wg_cmp, Wg_slc, Wg_swa):
    qh = q.reshape(B, SEQ, H, D)
    kh = k.reshape(B, SEQ, H, D)
    vh = v.reshape(B, SEQ, H, D)

    out4 = pl.pallas_call(
        _body,
        grid=(B,),
        in_specs=[
            pl.BlockSpec((1, SEQ, H, D), lambda b: (b, 0, 0, 0)),
            pl.BlockSpec((1, SEQ, H, D), lambda b: (b, 0, 0, 0)),
            pl.BlockSpec((1, SEQ, H, D), lambda b: (b, 0, 0, 0)),
            pl.BlockSpec((1, D), lambda b: (0, 0)),
            pl.BlockSpec((1, D), lambda b: (0, 0)),
        ],
        out_specs=pl.BlockSpec((1, SEQ, H, D), lambda b: (b, 0, 0, 0)),
        out_shape=jax.ShapeDtypeStruct((B, SEQ, H, D), jnp.float32),
        compiler_params=pltpu.CompilerParams(
            dimension_semantics=("parallel",)),
    )(qh, kh, vh, Wg_cmp.reshape(1, D), Wg_slc.reshape(1, D))

    return out4.reshape(T, H, D)
